# baseline (speedup 1.0000x reference)
import functools

import jax
import jax.numpy as jnp
from jax import lax
from jax.experimental import pallas as pl
from jax.experimental.pallas import tpu as pltpu
from jax.experimental.pallas import tpu_sc as plsc

NW = 32
BLK = 8
ROW_SHIFT = 9


def _pad16(k: int) -> int:
    return ((k + 15) // 16) * 16


def _scal(v, k):
    return v[pl.ds(k, 16)][0]


def _copy_block(src_hbm, dst_hbm, e0, elems, ch, cb0, cb1, sr0, sr1,
                sw0, sw1):
    nit = elems // (2 * ch)

    def it(i, carry):
        bb = e0 + i * 2 * ch

        @pl.when(i > 0)
        def _():
            pltpu.make_async_copy(
                cb0, dst_hbm.at[pl.ds(bb - 2 * ch, ch)], sw0).wait()
            pltpu.make_async_copy(
                cb1, dst_hbm.at[pl.ds(bb - ch, ch)], sw1).wait()

        pltpu.make_async_copy(src_hbm.at[pl.ds(bb, ch)], cb0, sr0).start()
        pltpu.make_async_copy(src_hbm.at[pl.ds(bb + ch, ch)], cb1, sr1).start()
        pltpu.make_async_copy(src_hbm.at[pl.ds(bb, ch)], cb0, sr0).wait()
        pltpu.make_async_copy(cb0, dst_hbm.at[pl.ds(bb, ch)], sw0).start()
        pltpu.make_async_copy(src_hbm.at[pl.ds(bb + ch, ch)], cb1, sr1).wait()
        pltpu.make_async_copy(cb1, dst_hbm.at[pl.ds(bb + ch, ch)], sw1).start()
        return carry

    lax.fori_loop(0, nit, it, 0)
    last = e0 + elems - 2 * ch
    pltpu.make_async_copy(cb0, dst_hbm.at[pl.ds(last, ch)], sw0).wait()
    pltpu.make_async_copy(cb1, dst_hbm.at[pl.ds(last + ch, ch)], sw1).wait()


def _y_kernel_body(B, C, C_PAD, NP, ROWS_W,
                   y_hbm, r_hbm, t_hbm, tp_hbm, rp_hbm,
                   rety_hbm,
                   r_v, t_v, tp_v, rp_v,
                   at_v, ar_v, atp_v, sr_v, srp_v,
                   pb, rg, wb, yb, ywb, cb0, cb1,
                   sem_a, sem_b, sem_c,
                   scp0, scp1, scp2, scp3):
    w = lax.axis_index("c") * 16 + lax.axis_index("s")
    row0 = w * ROWS_W

    mcps = [(r_hbm, r_v, scp0), (t_hbm, t_v, scp1), (tp_hbm, tp_v, scp2),
            (rp_hbm, rp_v, scp3)]
    for hh, vv, ss in mcps:
        pltpu.make_async_copy(hh, vv, ss).start()
    for hh, vv, ss in mcps:
        pltpu.make_async_copy(hh, vv, ss).wait()

    def scan_chunk(c16, carry):
        ca, cs = carry
        c = c16 * 16
        tv = t_v[pl.ds(c, 16)]
        rv = r_v[pl.ds(c, 16)]
        ma = lax.shift_right_logical(tv, ROW_SHIFT) == w
        ms = lax.shift_right_logical(rv, ROW_SHIFT) == w
        plsc.store_compressed(at_v.at[pl.ds(ca, 16)], tv, mask=ma)
        plsc.store_compressed(ar_v.at[pl.ds(ca, 16)], rv, mask=ma)
        plsc.store_compressed(atp_v.at[pl.ds(ca, 16)], tp_v[pl.ds(c, 16)],
                              mask=ma)
        plsc.store_compressed(sr_v.at[pl.ds(cs, 16)], rv, mask=ms)
        plsc.store_compressed(srp_v.at[pl.ds(cs, 16)], rp_v[pl.ds(c, 16)],
                              mask=ms)
        ca = ca + plsc.all_reduce_population_count(ma)[0]
        cs = cs + plsc.all_reduce_population_count(ms)[0]
        return (ca, cs)

    cnt_a, cnt_s = lax.fori_loop(0, NP // 16, scan_chunk,
                                 (jnp.int32(0), jnp.int32(0)))

    _copy_block(y_hbm, rety_hbm, row0, ROWS_W, 16, cb0, cb1,
                scp0, scp1, scp2, scp3)

    def a_prist(k, j):
        s = _scal(ar_v, k)
        return pltpu.make_async_copy(y_hbm.at[s],
                                     pb[j].at[pl.ds(0, C)], sem_a[j])

    def a_rmw(k, j):
        d = _scal(at_v, k)
        return pltpu.make_async_copy(rety_hbm.at[d],
                                     rg[j].at[pl.ds(0, C)], sem_b[j])

    def a_write(d, j):
        return pltpu.make_async_copy(wb[j].at[pl.ds(0, C)],
                                     rety_hbm.at[d], sem_c[j])

    for j in range(BLK):
        @pl.when(j < cnt_a)
        def _(j=j):
            a_prist(j, j).start()
            a_rmw(j, j).start()

    nblk_a = (cnt_a + BLK - 1) // BLK

    def a_blk(i, carry):
        pend = list(carry[:BLK])
        pdst = list(carry[BLK:2 * BLK])
        pprev = list(carry[2 * BLK:])
        for j in range(BLK):
            k = i * BLK + j
            alive = k < cnt_a
            tk = _scal(at_v, k)
            tpk = _scal(atp_v, k)
            dirty = jnp.zeros((), jnp.bool_)
            for jj in range(BLK):
                dirty = jnp.logical_or(dirty, pdst[jj] == tk)
                dirty = jnp.logical_or(dirty, pprev[jj] == tk)

            @pl.when(alive)
            def _(k=k, j=j):
                a_rmw(k, j).wait()
                a_prist(k, j).wait()

            for jj in range(BLK):
                @pl.when(jnp.logical_and(
                        alive, jnp.logical_and(pend[jj] == 1, pdst[jj] == tk)))
                def _(jj=jj):
                    a_write(pdst[jj], jj).wait()
                pend[jj] = jnp.where(
                    jnp.logical_and(alive, pdst[jj] == tk),
                    jnp.int32(0), pend[jj])

            @pl.when(jnp.logical_and(alive, dirty))
            def _(j=j, tk=tk):
                pltpu.sync_copy(rety_hbm.at[tk], rg[j].at[pl.ds(0, C)])

            @pl.when(jnp.logical_and(alive, pend[j] == 1))
            def _(j=j):
                a_write(pdst[j], j).wait()
            pend[j] = jnp.where(alive, jnp.int32(0), pend[j])

            @pl.when(alive)
            def _(k=k, j=j, tk=tk, tpk=tpk):
                @pl.loop(0, C_PAD, step=16)
                def _(c):
                    wb[j][pl.ds(c, 16)] = (rg[j][pl.ds(c, 16)]
                                           + pb[j][pl.ds(c, 16)] * tpk)

                a_write(tk, j).start()
                kn = k + BLK

                @pl.when(kn < cnt_a)
                def _():
                    a_prist(kn, j).start()
                    a_rmw(kn, j).start()

            pprev[j] = jnp.where(alive, pdst[j], pprev[j])
            pend[j] = jnp.where(alive, jnp.int32(1), pend[j])
            pdst[j] = jnp.where(alive, tk, pdst[j])
        return tuple(pend) + tuple(pdst) + tuple(pprev)

    carry0 = tuple(jnp.int32(0) for _ in range(BLK)) + \
        tuple(jnp.int32(-1) for _ in range(2 * BLK))
    carry = lax.fori_loop(0, nblk_a, a_blk, carry0)
    for j in range(BLK):
        @pl.when(carry[j] == 1)
        def _(j=j):
            a_write(carry[BLK + j], j).wait()

    def s_ygath(k, j):
        s = _scal(sr_v, k)
        return pltpu.make_async_copy(y_hbm.at[s],
                                     yb[j].at[pl.ds(0, C)], sem_a[j])

    def s_ywrite(d, j):
        return pltpu.make_async_copy(ywb[j].at[pl.ds(0, C)],
                                     rety_hbm.at[d], sem_c[j])

    for j in range(BLK):
        @pl.when(j < cnt_s)
        def _(j=j):
            s_ygath(j, j).start()

    nblk_s = (cnt_s + BLK - 1) // BLK

    def s_blk(i, carry):
        pend = list(carry[:BLK])
        pdst = list(carry[BLK:])
        for j in range(BLK):
            k = i * BLK + j
            alive = k < cnt_s
            rk = _scal(sr_v, k)
            rpk = _scal(srp_v, k)

            @pl.when(alive)
            def _(k=k, j=j):
                s_ygath(k, j).wait()

            for jj in range(BLK):
                cond = jnp.logical_and(
                    alive, jnp.logical_and(
                        pend[jj] == 1,
                        jnp.logical_or(pdst[jj] == rk, jj == j)))

                @pl.when(cond)
                def _(jj=jj):
                    s_ywrite(pdst[jj], jj).wait()
                pend[jj] = jnp.where(cond, jnp.int32(0), pend[jj])

            @pl.when(alive)
            def _(k=k, j=j, rk=rk, rpk=rpk):
                @pl.loop(0, C_PAD, step=16)
                def _(c):
                    ywb[j][pl.ds(c, 16)] = yb[j][pl.ds(c, 16)] * rpk

                s_ywrite(rk, j).start()
                kn = k + BLK

                @pl.when(kn < cnt_s)
                def _():
                    s_ygath(kn, j).start()

            pend[j] = jnp.where(alive, jnp.int32(1), pend[j])
            pdst[j] = jnp.where(alive, rk, pdst[j])
        return tuple(pend) + tuple(pdst)

    carry0_s = tuple(jnp.int32(0) for _ in range(BLK)) + \
        tuple(jnp.int32(-1) for _ in range(BLK))
    carry = lax.fori_loop(0, nblk_s, s_blk, carry0_s)
    for j in range(BLK):
        @pl.when(carry[j] == 1)
        def _(j=j):
            s_ywrite(carry[BLK + j], j).wait()


def _x_kernel_body(B, D, NP, ROWS_W,
                   x_hbm, r_hbm, t_hbm, mag_hbm,
                   ret_hbm,
                   r_v, t_v, mag_v,
                   sr_v, st_v, smag_v,
                   xtb, xrb, xwb, cbx0, cbx1,
                   sem_b, sem_d, sem_e,
                   scp0, scp1, scp2, scp3):
    w = lax.axis_index("c") * 16 + lax.axis_index("s")
    row0 = w * ROWS_W

    mcps = [(r_hbm, r_v, scp0), (t_hbm, t_v, scp1), (mag_hbm, mag_v, scp2)]
    for hh, vv, ss in mcps:
        pltpu.make_async_copy(hh, vv, ss).start()
    for hh, vv, ss in mcps:
        pltpu.make_async_copy(hh, vv, ss).wait()

    def scan_chunk(c16, cs):
        c = c16 * 16
        rv = r_v[pl.ds(c, 16)]
        ms = lax.shift_right_logical(rv, ROW_SHIFT) == w
        plsc.store_compressed(sr_v.at[pl.ds(cs, 16)], rv, mask=ms)
        plsc.store_compressed(st_v.at[pl.ds(cs, 16)], t_v[pl.ds(c, 16)],
                              mask=ms)
        plsc.store_compressed(smag_v.at[pl.ds(cs, 16)], mag_v[pl.ds(c, 16)],
                              mask=ms)
        return cs + plsc.all_reduce_population_count(ms)[0]

    cnt_s = lax.fori_loop(0, NP // 16, scan_chunk, jnp.int32(0))

    _copy_block(x_hbm, ret_hbm, row0, ROWS_W, 64, cbx0, cbx1,
                scp0, scp1, scp2, scp3)

    def s_xt(k, j):
        s = _scal(st_v, k)
        return pltpu.make_async_copy(x_hbm.at[pl.ds(s, 1)], xtb[j],
                                     sem_b[j])

    def s_xr(k, j):
        s = _scal(sr_v, k)
        return pltpu.make_async_copy(x_hbm.at[pl.ds(s, 1)], xrb[j],
                                     sem_d[j])

    def s_xwrite(d, j):
        return pltpu.make_async_copy(xwb[j], ret_hbm.at[pl.ds(d, 1)],
                                     sem_e[j])

    for j in range(BLK):
        @pl.when(j < cnt_s)
        def _(j=j):
            s_xt(j, j).start()
            s_xr(j, j).start()

    nblk_s = (cnt_s + BLK - 1) // BLK

    def s_blk(i, carry):
        pend = list(carry[:BLK])
        pdst = list(carry[BLK:])
        for j in range(BLK):
            k = i * BLK + j
            alive = k < cnt_s
            rk = _scal(sr_v, k)
            magk = _scal(smag_v, k)

            @pl.when(alive)
            def _(k=k, j=j):
                s_xt(k, j).wait()
                s_xr(k, j).wait()

            for jj in range(BLK):
                cond = jnp.logical_and(
                    alive, jnp.logical_and(
                        pend[jj] == 1,
                        jnp.logical_or(pdst[jj] == rk, jj == j)))

                @pl.when(cond)
                def _(jj=jj):
                    s_xwrite(pdst[jj], jj).wait()
                pend[jj] = jnp.where(cond, jnp.int32(0), pend[jj])

            @pl.when(alive)
            def _(k=k, j=j, rk=rk, magk=magk):
                om = 1.0 - magk

                @pl.loop(0, D, step=16)
                def _(c):
                    xwb[j][0, pl.ds(c, 16)] = (xtb[j][0, pl.ds(c, 16)] * magk
                                               + xrb[j][0, pl.ds(c, 16)] * om)

                s_xwrite(rk, j).start()
                kn = k + BLK

                @pl.when(kn < cnt_s)
                def _():
                    s_xt(kn, j).start()
                    s_xr(kn, j).start()

            pend[j] = jnp.where(alive, jnp.int32(1), pend[j])
            pdst[j] = jnp.where(alive, rk, pdst[j])
        return tuple(pend) + tuple(pdst)

    carry0_s = tuple(jnp.int32(0) for _ in range(BLK)) + \
        tuple(jnp.int32(-1) for _ in range(BLK))
    carry = lax.fori_loop(0, nblk_s, s_blk, carry0_s)
    for j in range(BLK):
        @pl.when(carry[j] == 1)
        def _(j=j):
            s_xwrite(carry[BLK + j], j).wait()


@jax.jit
def kernel(x, y, ref_index, target_index, mag):
    B, D = x.shape
    C = y.shape[1]
    n = ref_index.shape[0]
    C_PAD = _pad16(C)
    NP = _pad16(n)
    ROWS_W = B // NW

    r = ref_index.astype(jnp.int32)
    t = target_index.astype(jnp.int32)
    am = jnp.abs(mag)
    a1m = jnp.abs(1.0 - mag)
    tot = am + a1m
    tp = am / tot
    rp = a1m / tot

    pad = NP - n
    big = jnp.int32(1 << 20)

    def pi(a):
        return jnp.pad(a, (0, pad), constant_values=big)

    def pf(a):
        return jnp.pad(a, (0, pad))

    mesh = plsc.VectorSubcoreMesh(core_axis_name="c", subcore_axis_name="s")
    NPS = NP + 16
    cp = pltpu.CompilerParams(use_tc_tiling_on_sc=False,
                              needs_layout_passes=False)

    y_body = functools.partial(_y_kernel_body, B, C, C_PAD, NP, ROWS_W)
    f_y = pl.kernel(
        y_body,
        out_type=jax.ShapeDtypeStruct((B, C), jnp.float32),
        mesh=mesh,
        scratch_types=[
            pltpu.VMEM((NP,), jnp.int32),
            pltpu.VMEM((NP,), jnp.int32),
            pltpu.VMEM((NP,), jnp.float32),
            pltpu.VMEM((NP,), jnp.float32),
            pltpu.VMEM((NPS,), jnp.int32),
            pltpu.VMEM((NPS,), jnp.int32),
            pltpu.VMEM((NPS,), jnp.float32),
            pltpu.VMEM((NPS,), jnp.int32),
            pltpu.VMEM((NPS,), jnp.float32),
            [pltpu.VMEM((C_PAD,), jnp.float32) for _ in range(BLK)],
            [pltpu.VMEM((C_PAD,), jnp.float32) for _ in range(BLK)],
            [pltpu.VMEM((C_PAD,), jnp.float32) for _ in range(BLK)],
            [pltpu.VMEM((C_PAD,), jnp.float32) for _ in range(BLK)],
            [pltpu.VMEM((C_PAD,), jnp.float32) for _ in range(BLK)],
            pltpu.VMEM((16, C), jnp.float32),
            pltpu.VMEM((16, C), jnp.float32),
            [pltpu.SemaphoreType.DMA for _ in range(BLK)],
            [pltpu.SemaphoreType.DMA for _ in range(BLK)],
            [pltpu.SemaphoreType.DMA for _ in range(BLK)],
            pltpu.SemaphoreType.DMA,
            pltpu.SemaphoreType.DMA,
            pltpu.SemaphoreType.DMA,
            pltpu.SemaphoreType.DMA,
        ],
        compiler_params=cp,
    )

    cp_x = pltpu.CompilerParams(needs_layout_passes=False)
    x_body = functools.partial(_x_kernel_body, B, D, NP, ROWS_W)
    f_x = pl.kernel(
        x_body,
        out_type=jax.ShapeDtypeStruct((B, D), jnp.float32),
        mesh=mesh,
        scratch_types=[
            pltpu.VMEM((NP,), jnp.int32),
            pltpu.VMEM((NP,), jnp.int32),
            pltpu.VMEM((NP,), jnp.float32),
            pltpu.VMEM((NPS,), jnp.int32),
            pltpu.VMEM((NPS,), jnp.int32),
            pltpu.VMEM((NPS,), jnp.float32),
            [pltpu.VMEM((1, D), jnp.float32) for _ in range(BLK)],
            [pltpu.VMEM((1, D), jnp.float32) for _ in range(BLK)],
            [pltpu.VMEM((1, D), jnp.float32) for _ in range(BLK)],
            pltpu.VMEM((64, D), jnp.float32),
            pltpu.VMEM((64, D), jnp.float32),
            [pltpu.SemaphoreType.DMA for _ in range(BLK)],
            [pltpu.SemaphoreType.DMA for _ in range(BLK)],
            [pltpu.SemaphoreType.DMA for _ in range(BLK)],
            pltpu.SemaphoreType.DMA,
            pltpu.SemaphoreType.DMA,
            pltpu.SemaphoreType.DMA,
            pltpu.SemaphoreType.DMA,
        ],
        compiler_params=cp_x,
    )

    ri, ti = pi(r), pi(t)
    ret = f_x(x, ri, ti, pf(mag))
    ret_y = f_y(y, ri, ti, pf(tp), pf(rp))
    return (ret, ret_y)


# speedup vs baseline: 1.0257x; 1.0257x over previous
import functools

import jax
import jax.numpy as jnp
from jax import lax
from jax.experimental import pallas as pl
from jax.experimental.pallas import tpu as pltpu
from jax.experimental.pallas import tpu_sc as plsc

NW = 32
BLK = 4
ROW_SHIFT = 9


def _pad16(k: int) -> int:
    return ((k + 15) // 16) * 16


def _scal(v, k):
    return v[pl.ds(k, 16)][0]


def _copy_block(src_hbm, dst_hbm, e0, elems, ch, cb0, cb1, sr0, sr1,
                sw0, sw1):
    nit = elems // (2 * ch)

    def it(i, carry):
        bb = e0 + i * 2 * ch

        @pl.when(i > 0)
        def _():
            pltpu.make_async_copy(
                cb0, dst_hbm.at[pl.ds(bb - 2 * ch, ch)], sw0).wait()
            pltpu.make_async_copy(
                cb1, dst_hbm.at[pl.ds(bb - ch, ch)], sw1).wait()

        pltpu.make_async_copy(src_hbm.at[pl.ds(bb, ch)], cb0, sr0).start()
        pltpu.make_async_copy(src_hbm.at[pl.ds(bb + ch, ch)], cb1, sr1).start()
        pltpu.make_async_copy(src_hbm.at[pl.ds(bb, ch)], cb0, sr0).wait()
        pltpu.make_async_copy(cb0, dst_hbm.at[pl.ds(bb, ch)], sw0).start()
        pltpu.make_async_copy(src_hbm.at[pl.ds(bb + ch, ch)], cb1, sr1).wait()
        pltpu.make_async_copy(cb1, dst_hbm.at[pl.ds(bb + ch, ch)], sw1).start()
        return carry

    lax.fori_loop(0, nit, it, 0)
    last = e0 + elems - 2 * ch
    pltpu.make_async_copy(cb0, dst_hbm.at[pl.ds(last, ch)], sw0).wait()
    pltpu.make_async_copy(cb1, dst_hbm.at[pl.ds(last + ch, ch)], sw1).wait()


def _y_kernel_body(B, C, C_PAD, NP, ROWS_W,
                   y_hbm, r_hbm, t_hbm, tp_hbm, rp_hbm,
                   rety_hbm,
                   r_v, t_v, tp_v, rp_v,
                   at_v, ar_v, atp_v, sr_v, srp_v,
                   pb, rg, wb, yb, ywb, cb0, cb1,
                   sem_a, sem_b, sem_c,
                   scp0, scp1, scp2, scp3):
    w = lax.axis_index("c") * 16 + lax.axis_index("s")
    row0 = w * ROWS_W

    mcps = [(r_hbm, r_v, scp0), (t_hbm, t_v, scp1), (tp_hbm, tp_v, scp2),
            (rp_hbm, rp_v, scp3)]
    for hh, vv, ss in mcps:
        pltpu.make_async_copy(hh, vv, ss).start()
    for hh, vv, ss in mcps:
        pltpu.make_async_copy(hh, vv, ss).wait()

    def scan_chunk(c16, carry):
        ca, cs = carry
        c = c16 * 16
        tv = t_v[pl.ds(c, 16)]
        rv = r_v[pl.ds(c, 16)]
        ma = lax.shift_right_logical(tv, ROW_SHIFT) == w
        ms = lax.shift_right_logical(rv, ROW_SHIFT) == w
        plsc.store_compressed(at_v.at[pl.ds(ca, 16)], tv, mask=ma)
        plsc.store_compressed(ar_v.at[pl.ds(ca, 16)], rv, mask=ma)
        plsc.store_compressed(atp_v.at[pl.ds(ca, 16)], tp_v[pl.ds(c, 16)],
                              mask=ma)
        plsc.store_compressed(sr_v.at[pl.ds(cs, 16)], rv, mask=ms)
        plsc.store_compressed(srp_v.at[pl.ds(cs, 16)], rp_v[pl.ds(c, 16)],
                              mask=ms)
        ca = ca + plsc.all_reduce_population_count(ma)[0]
        cs = cs + plsc.all_reduce_population_count(ms)[0]
        return (ca, cs)

    cnt_a, cnt_s = lax.fori_loop(0, NP // 16, scan_chunk,
                                 (jnp.int32(0), jnp.int32(0)))

    _copy_block(y_hbm, rety_hbm, row0, ROWS_W, 32, cb0, cb1,
                scp0, scp1, scp2, scp3)

    def a_prist(k, j):
        s = _scal(ar_v, k)
        return pltpu.make_async_copy(y_hbm.at[s],
                                     pb[j].at[pl.ds(0, C)], sem_a[j])

    def a_rmw(k, j):
        d = _scal(at_v, k)
        return pltpu.make_async_copy(rety_hbm.at[d],
                                     rg[j].at[pl.ds(0, C)], sem_b[j])

    def a_write(d, j):
        return pltpu.make_async_copy(wb[j].at[pl.ds(0, C)],
                                     rety_hbm.at[d], sem_c[j])

    for j in range(BLK):
        @pl.when(j < cnt_a)
        def _(j=j):
            a_prist(j, j).start()
            a_rmw(j, j).start()

    nblk_a = (cnt_a + BLK - 1) // BLK

    def a_blk(i, carry):
        pend = list(carry[:BLK])
        pdst = list(carry[BLK:2 * BLK])
        pprev = list(carry[2 * BLK:])
        for j in range(BLK):
            k = i * BLK + j
            alive = k < cnt_a
            tk = _scal(at_v, k)
            tpk = _scal(atp_v, k)
            dirty = jnp.zeros((), jnp.bool_)
            for jj in range(BLK):
                dirty = jnp.logical_or(dirty, pdst[jj] == tk)
                dirty = jnp.logical_or(dirty, pprev[jj] == tk)

            @pl.when(alive)
            def _(k=k, j=j):
                a_rmw(k, j).wait()
                a_prist(k, j).wait()

            for jj in range(BLK):
                @pl.when(jnp.logical_and(
                        alive, jnp.logical_and(pend[jj] == 1, pdst[jj] == tk)))
                def _(jj=jj):
                    a_write(pdst[jj], jj).wait()
                pend[jj] = jnp.where(
                    jnp.logical_and(alive, pdst[jj] == tk),
                    jnp.int32(0), pend[jj])

            @pl.when(jnp.logical_and(alive, dirty))
            def _(j=j, tk=tk):
                pltpu.sync_copy(rety_hbm.at[tk], rg[j].at[pl.ds(0, C)])

            @pl.when(jnp.logical_and(alive, pend[j] == 1))
            def _(j=j):
                a_write(pdst[j], j).wait()
            pend[j] = jnp.where(alive, jnp.int32(0), pend[j])

            @pl.when(alive)
            def _(k=k, j=j, tk=tk, tpk=tpk):
                @pl.loop(0, C_PAD, step=16)
                def _(c):
                    wb[j][pl.ds(c, 16)] = (rg[j][pl.ds(c, 16)]
                                           + pb[j][pl.ds(c, 16)] * tpk)

                a_write(tk, j).start()
                kn = k + BLK

                @pl.when(kn < cnt_a)
                def _():
                    a_prist(kn, j).start()
                    a_rmw(kn, j).start()

            pprev[j] = jnp.where(alive, pdst[j], pprev[j])
            pend[j] = jnp.where(alive, jnp.int32(1), pend[j])
            pdst[j] = jnp.where(alive, tk, pdst[j])
        return tuple(pend) + tuple(pdst) + tuple(pprev)

    carry0 = tuple(jnp.int32(0) for _ in range(BLK)) + \
        tuple(jnp.int32(-1) for _ in range(2 * BLK))
    carry = lax.fori_loop(0, nblk_a, a_blk, carry0)
    for j in range(BLK):
        @pl.when(carry[j] == 1)
        def _(j=j):
            a_write(carry[BLK + j], j).wait()

    def s_ygath(k, j):
        s = _scal(sr_v, k)
        return pltpu.make_async_copy(y_hbm.at[s],
                                     yb[j].at[pl.ds(0, C)], sem_a[j])

    def s_ywrite(d, j):
        return pltpu.make_async_copy(ywb[j].at[pl.ds(0, C)],
                                     rety_hbm.at[d], sem_c[j])

    for j in range(BLK):
        @pl.when(j < cnt_s)
        def _(j=j):
            s_ygath(j, j).start()

    nblk_s = (cnt_s + BLK - 1) // BLK

    def s_blk(i, carry):
        pend = list(carry[:BLK])
        pdst = list(carry[BLK:])
        for j in range(BLK):
            k = i * BLK + j
            alive = k < cnt_s
            rk = _scal(sr_v, k)
            rpk = _scal(srp_v, k)

            @pl.when(alive)
            def _(k=k, j=j):
                s_ygath(k, j).wait()

            for jj in range(BLK):
                cond = jnp.logical_and(
                    alive, jnp.logical_and(
                        pend[jj] == 1,
                        jnp.logical_or(pdst[jj] == rk, jj == j)))

                @pl.when(cond)
                def _(jj=jj):
                    s_ywrite(pdst[jj], jj).wait()
                pend[jj] = jnp.where(cond, jnp.int32(0), pend[jj])

            @pl.when(alive)
            def _(k=k, j=j, rk=rk, rpk=rpk):
                @pl.loop(0, C_PAD, step=16)
                def _(c):
                    ywb[j][pl.ds(c, 16)] = yb[j][pl.ds(c, 16)] * rpk

                s_ywrite(rk, j).start()
                kn = k + BLK

                @pl.when(kn < cnt_s)
                def _():
                    s_ygath(kn, j).start()

            pend[j] = jnp.where(alive, jnp.int32(1), pend[j])
            pdst[j] = jnp.where(alive, rk, pdst[j])
        return tuple(pend) + tuple(pdst)

    carry0_s = tuple(jnp.int32(0) for _ in range(BLK)) + \
        tuple(jnp.int32(-1) for _ in range(BLK))
    carry = lax.fori_loop(0, nblk_s, s_blk, carry0_s)
    for j in range(BLK):
        @pl.when(carry[j] == 1)
        def _(j=j):
            s_ywrite(carry[BLK + j], j).wait()


def _x_kernel_body(B, D, NP, ROWS_W,
                   x_hbm, r_hbm, t_hbm, mag_hbm,
                   ret_hbm,
                   r_v, t_v, mag_v,
                   sr_v, st_v, smag_v,
                   xtb, xrb, xwb, cbx0, cbx1,
                   sem_b, sem_d, sem_e,
                   scp0, scp1, scp2, scp3):
    w = lax.axis_index("c") * 16 + lax.axis_index("s")
    row0 = w * ROWS_W

    mcps = [(r_hbm, r_v, scp0), (t_hbm, t_v, scp1), (mag_hbm, mag_v, scp2)]
    for hh, vv, ss in mcps:
        pltpu.make_async_copy(hh, vv, ss).start()
    for hh, vv, ss in mcps:
        pltpu.make_async_copy(hh, vv, ss).wait()

    def scan_chunk(c16, cs):
        c = c16 * 16
        rv = r_v[pl.ds(c, 16)]
        ms = lax.shift_right_logical(rv, ROW_SHIFT) == w
        plsc.store_compressed(sr_v.at[pl.ds(cs, 16)], rv, mask=ms)
        plsc.store_compressed(st_v.at[pl.ds(cs, 16)], t_v[pl.ds(c, 16)],
                              mask=ms)
        plsc.store_compressed(smag_v.at[pl.ds(cs, 16)], mag_v[pl.ds(c, 16)],
                              mask=ms)
        return cs + plsc.all_reduce_population_count(ms)[0]

    cnt_s = lax.fori_loop(0, NP // 16, scan_chunk, jnp.int32(0))

    _copy_block(x_hbm, ret_hbm, row0, ROWS_W, 64, cbx0, cbx1,
                scp0, scp1, scp2, scp3)

    def s_xt(k, j):
        s = _scal(st_v, k)
        return pltpu.make_async_copy(x_hbm.at[pl.ds(s, 1)], xtb[j],
                                     sem_b[j])

    def s_xr(k, j):
        s = _scal(sr_v, k)
        return pltpu.make_async_copy(x_hbm.at[pl.ds(s, 1)], xrb[j],
                                     sem_d[j])

    def s_xwrite(d, j):
        return pltpu.make_async_copy(xwb[j], ret_hbm.at[pl.ds(d, 1)],
                                     sem_e[j])

    for j in range(BLK):
        @pl.when(j < cnt_s)
        def _(j=j):
            s_xt(j, j).start()
            s_xr(j, j).start()

    nblk_s = (cnt_s + BLK - 1) // BLK

    def s_blk(i, carry):
        pend = list(carry[:BLK])
        pdst = list(carry[BLK:])
        for j in range(BLK):
            k = i * BLK + j
            alive = k < cnt_s
            rk = _scal(sr_v, k)
            magk = _scal(smag_v, k)

            @pl.when(alive)
            def _(k=k, j=j):
                s_xt(k, j).wait()
                s_xr(k, j).wait()

            for jj in range(BLK):
                cond = jnp.logical_and(
                    alive, jnp.logical_and(
                        pend[jj] == 1,
                        jnp.logical_or(pdst[jj] == rk, jj == j)))

                @pl.when(cond)
                def _(jj=jj):
                    s_xwrite(pdst[jj], jj).wait()
                pend[jj] = jnp.where(cond, jnp.int32(0), pend[jj])

            @pl.when(alive)
            def _(k=k, j=j, rk=rk, magk=magk):
                om = 1.0 - magk

                @pl.loop(0, D, step=16)
                def _(c):
                    xwb[j][0, pl.ds(c, 16)] = (xtb[j][0, pl.ds(c, 16)] * magk
                                               + xrb[j][0, pl.ds(c, 16)] * om)

                s_xwrite(rk, j).start()
                kn = k + BLK

                @pl.when(kn < cnt_s)
                def _():
                    s_xt(kn, j).start()
                    s_xr(kn, j).start()

            pend[j] = jnp.where(alive, jnp.int32(1), pend[j])
            pdst[j] = jnp.where(alive, rk, pdst[j])
        return tuple(pend) + tuple(pdst)

    carry0_s = tuple(jnp.int32(0) for _ in range(BLK)) + \
        tuple(jnp.int32(-1) for _ in range(BLK))
    carry = lax.fori_loop(0, nblk_s, s_blk, carry0_s)
    for j in range(BLK):
        @pl.when(carry[j] == 1)
        def _(j=j):
            s_xwrite(carry[BLK + j], j).wait()


@jax.jit
def kernel(x, y, ref_index, target_index, mag):
    B, D = x.shape
    C = y.shape[1]
    n = ref_index.shape[0]
    C_PAD = _pad16(C)
    NP = _pad16(n)
    ROWS_W = B // NW

    r = ref_index.astype(jnp.int32)
    t = target_index.astype(jnp.int32)
    am = jnp.abs(mag)
    a1m = jnp.abs(1.0 - mag)
    tot = am + a1m
    tp = am / tot
    rp = a1m / tot

    pad = NP - n
    big = jnp.int32(1 << 20)

    def pi(a):
        return jnp.pad(a, (0, pad), constant_values=big)

    def pf(a):
        return jnp.pad(a, (0, pad))

    mesh = plsc.VectorSubcoreMesh(core_axis_name="c", subcore_axis_name="s")
    NPS = NP + 16
    cp = pltpu.CompilerParams(use_tc_tiling_on_sc=False,
                              needs_layout_passes=False)

    y_body = functools.partial(_y_kernel_body, B, C, C_PAD, NP, ROWS_W)
    f_y = pl.kernel(
        y_body,
        out_type=jax.ShapeDtypeStruct((B, C), jnp.float32),
        mesh=mesh,
        scratch_types=[
            pltpu.VMEM((NP,), jnp.int32),
            pltpu.VMEM((NP,), jnp.int32),
            pltpu.VMEM((NP,), jnp.float32),
            pltpu.VMEM((NP,), jnp.float32),
            pltpu.VMEM((NPS,), jnp.int32),
            pltpu.VMEM((NPS,), jnp.int32),
            pltpu.VMEM((NPS,), jnp.float32),
            pltpu.VMEM((NPS,), jnp.int32),
            pltpu.VMEM((NPS,), jnp.float32),
            [pltpu.VMEM((C_PAD,), jnp.float32) for _ in range(BLK)],
            [pltpu.VMEM((C_PAD,), jnp.float32) for _ in range(BLK)],
            [pltpu.VMEM((C_PAD,), jnp.float32) for _ in range(BLK)],
            [pltpu.VMEM((C_PAD,), jnp.float32) for _ in range(BLK)],
            [pltpu.VMEM((C_PAD,), jnp.float32) for _ in range(BLK)],
            pltpu.VMEM((32, C), jnp.float32),
            pltpu.VMEM((32, C), jnp.float32),
            [pltpu.SemaphoreType.DMA for _ in range(BLK)],
            [pltpu.SemaphoreType.DMA for _ in range(BLK)],
            [pltpu.SemaphoreType.DMA for _ in range(BLK)],
            pltpu.SemaphoreType.DMA,
            pltpu.SemaphoreType.DMA,
            pltpu.SemaphoreType.DMA,
            pltpu.SemaphoreType.DMA,
        ],
        compiler_params=cp,
    )

    cp_x = pltpu.CompilerParams(needs_layout_passes=False)
    x_body = functools.partial(_x_kernel_body, B, D, NP, ROWS_W)
    f_x = pl.kernel(
        x_body,
        out_type=jax.ShapeDtypeStruct((B, D), jnp.float32),
        mesh=mesh,
        scratch_types=[
            pltpu.VMEM((NP,), jnp.int32),
            pltpu.VMEM((NP,), jnp.int32),
            pltpu.VMEM((NP,), jnp.float32),
            pltpu.VMEM((NPS,), jnp.int32),
            pltpu.VMEM((NPS,), jnp.int32),
            pltpu.VMEM((NPS,), jnp.float32),
            [pltpu.VMEM((1, D), jnp.float32) for _ in range(BLK)],
            [pltpu.VMEM((1, D), jnp.float32) for _ in range(BLK)],
            [pltpu.VMEM((1, D), jnp.float32) for _ in range(BLK)],
            pltpu.VMEM((64, D), jnp.float32),
            pltpu.VMEM((64, D), jnp.float32),
            [pltpu.SemaphoreType.DMA for _ in range(BLK)],
            [pltpu.SemaphoreType.DMA for _ in range(BLK)],
            [pltpu.SemaphoreType.DMA for _ in range(BLK)],
            pltpu.SemaphoreType.DMA,
            pltpu.SemaphoreType.DMA,
            pltpu.SemaphoreType.DMA,
            pltpu.SemaphoreType.DMA,
        ],
        compiler_params=cp_x,
    )

    ri, ti = pi(r), pi(t)
    ret = f_x(x, ri, ti, pf(mag))
    ret_y = f_y(y, ri, ti, pf(tp), pf(rp))
    return (ret, ret_y)
